# baseline (speedup 1.0000x reference)
import functools

import jax
import jax.numpy as jnp
from jax import lax
from jax.experimental import pallas as pl
from jax.experimental.pallas import tpu as pltpu
from jax.experimental.pallas import tpu_sc as plsc

_NBUF = 3
_K = 40
_UNROLL = 8


def _make_gather(V, D, B):
    info = plsc.get_sparse_core_info()
    NC, NS = info.num_cores, info.num_subcores
    NW = NC * NS
    assert B % NW == 0
    b_per_w = B // NW
    assert b_per_w % 8 == 0
    K = _K
    Dp = D // 2
    n_chunks = (b_per_w // K) // _NBUF * _NBUF
    n_rounds = n_chunks // _NBUF
    n_left = b_per_w // K - n_chunks
    assert b_per_w % K == 0 and n_rounds >= 3
    n_vec = K * Dp // 16
    assert n_vec % _UNROLL == 0

    mesh = plsc.VectorSubcoreMesh(core_axis_name="c", subcore_axis_name="s")

    @functools.partial(
        pl.kernel,
        mesh=mesh,
        out_type=jax.ShapeDtypeStruct((B * D,), jnp.float32),
        scratch_types=[
            pltpu.VMEM((b_per_w,), jnp.int32),
        ]
        + [pltpu.VMEM((K, Dp), jnp.int32) for _ in range(_NBUF)]
        + [pltpu.VMEM((K * D,), jnp.float32) for _ in range(_NBUF)]
        + [pltpu.SemaphoreType.DMA for _ in range(2 * _NBUF)],
    )
    def gather_kernel(table_hbm, idx_hbm, out_hbm, idx_v, *rest):
        ibufs = rest[:_NBUF]
        obufs = rest[_NBUF : 2 * _NBUF]
        gsems = rest[2 * _NBUF : 3 * _NBUF]
        osems = rest[3 * _NBUF :]
        wid = lax.axis_index("s") * NC + lax.axis_index("c")
        base = wid * b_per_w
        pltpu.sync_copy(idx_hbm.at[pl.ds(base, b_per_w)], idx_v)

        def start_gather(c, j):
            pltpu.async_copy(
                table_hbm.at[idx_v.at[pl.ds(c * K, K)]], ibufs[j], gsems[j]
            )

        def wait_gather(c, j):
            pltpu.make_async_copy(
                table_hbm.at[idx_v.at[pl.ds(c * K, K)]], ibufs[j], gsems[j]
            ).wait()

        def out_view(c, j):
            return pltpu.make_async_copy(
                obufs[j],
                out_hbm.at[pl.ds((base + c * K) * D, K * D)],
                osems[j],
            )

        def widen(j):
            src = ibufs[j]
            dst = obufs[j]
            mask = jnp.full((16,), -65536, jnp.int32)

            def body(r, carry):
                rb = r * D
                for cb in range(Dp // 16):
                    o = cb * 16
                    v = src[r, pl.ds(o, 16)]
                    a = lax.bitcast_convert_type(v << 16, jnp.float32)
                    b = lax.bitcast_convert_type(v & mask, jnp.float32)
                    dst[pl.ds(rb + 2 * o, 16)] = a
                    dst[pl.ds(rb + 2 * o + 16, 16)] = b
                return carry

            lax.fori_loop(0, K, body, 0)

        def step(c, j, first, issue_next):
            wait_gather(c, j)
            if not first:
                out_view(c - _NBUF, j).wait()
            widen(j)
            out_view(c, j).start()
            if issue_next:
                start_gather(c + _NBUF, j)

        for j in range(_NBUF):
            start_gather(j, j)
        for j in range(_NBUF):
            step(j, j, True, True)

        def body(i, carry):
            c0 = i * _NBUF
            for j in range(_NBUF):
                step(c0 + j, j, False, True)
            return carry

        lax.fori_loop(1, n_rounds - 1, body, 0)

        cl = (n_rounds - 1) * _NBUF
        for j in range(_NBUF):
            step(cl + j, j, False, j < n_left)
        for j in range(n_left):
            c = n_chunks + j
            wait_gather(c, j)
            out_view(cl + j, j).wait()
            widen(j)
            out_view(c, j).start()
        for j in range(n_left):
            out_view(n_chunks + j, j).wait()
        for j in range(n_left, _NBUF):
            out_view(cl + j, j).wait()

    return gather_kernel


def kernel(idx, table):
    V, D = table.shape
    orig_shape = idx.shape
    idx_flat = idx.reshape(-1).astype(jnp.int32)
    B = idx_flat.shape[0]
    tb = table.astype(jnp.bfloat16)
    tb = tb.reshape(V, D // 32, 2, 16).transpose(0, 1, 3, 2).reshape(V, D)
    tb = lax.bitcast_convert_type(tb.reshape(V, D // 2, 2), jnp.int32)
    out = _make_gather(V, D, B)(tb, idx_flat)
    return out.reshape(*orig_shape, D)


# speedup vs baseline: 1.3275x; 1.3275x over previous
import functools

import jax
import jax.numpy as jnp
from jax import lax
from jax.experimental import pallas as pl
from jax.experimental.pallas import tpu as pltpu
from jax.experimental.pallas import tpu_sc as plsc

_NBUF = 2
_K = 40


def _make_gather(V, D, B):
    info = plsc.get_sparse_core_info()
    NC, NS = info.num_cores, info.num_subcores
    NW = NC * NS
    assert B % NW == 0
    b_per_w = B // NW
    assert b_per_w % 8 == 0
    K = _K
    n_chunks = (b_per_w // K) // _NBUF * _NBUF
    n_rounds = n_chunks // _NBUF
    tail = b_per_w - n_chunks * K
    assert tail % 8 == 0 and tail <= K and n_rounds >= 3

    mesh = plsc.VectorSubcoreMesh(core_axis_name="c", subcore_axis_name="s")

    @functools.partial(
        pl.kernel,
        mesh=mesh,
        out_type=jax.ShapeDtypeStruct((B, D), jnp.float32),
        scratch_types=[
            pltpu.VMEM((b_per_w,), jnp.int32),
            pltpu.VMEM_SHARED((NS * _NBUF * K, D), jnp.float32),
        ]
        + [pltpu.VMEM((K, D), jnp.float32) for _ in range(_NBUF)]
        + [pltpu.SemaphoreType.DMA for _ in range(2 * _NBUF)],
    )
    def gather_kernel(table_hbm, idx_hbm, out_hbm, idx_v, sp, *rest):
        bufs = rest[:_NBUF]
        gsems = rest[_NBUF : 2 * _NBUF]
        hsems = rest[2 * _NBUF :]
        sid = lax.axis_index("s")
        wid = sid * NC + lax.axis_index("c")
        base = wid * b_per_w
        pltpu.sync_copy(idx_hbm.at[pl.ds(base, b_per_w)], idx_v)

        def slot(j, n=K):
            return sp.at[pl.ds((sid * _NBUF + j) * K, n)]

        def start_gather(c, j):
            pltpu.async_copy(
                table_hbm.at[idx_v.at[pl.ds(c * K, K)]], bufs[j], gsems[j]
            )

        def wait_gather(c, j):
            pltpu.make_async_copy(
                table_hbm.at[idx_v.at[pl.ds(c * K, K)]], bufs[j], gsems[j]
            ).wait()

        def start_hbm(c, j):
            pltpu.async_copy(
                slot(j), out_hbm.at[pl.ds(base + c * K, K)], hsems[j]
            )

        def wait_hbm(c, j):
            pltpu.make_async_copy(
                slot(j), out_hbm.at[pl.ds(base + c * K, K)], hsems[j]
            ).wait()

        def step(c, j, first, issue_next):
            wait_gather(c, j)
            if not first:
                wait_hbm(c - _NBUF, j)
            pltpu.sync_copy(bufs[j], slot(j))
            start_hbm(c, j)
            if issue_next:
                start_gather(c + _NBUF, j)

        for j in range(_NBUF):
            start_gather(j, j)
        for j in range(_NBUF):
            step(j, j, True, True)

        def body(i, carry):
            c0 = i * _NBUF
            for j in range(_NBUF):
                step(c0 + j, j, False, True)
            return carry

        lax.fori_loop(1, n_rounds - 1, body, 0)

        cl = (n_rounds - 1) * _NBUF
        for j in range(_NBUF):
            step(cl + j, j, False, False)
        if tail:
            toff = n_chunks * K
            tb = bufs[0].at[pl.ds(0, tail)]
            pltpu.async_copy(
                table_hbm.at[idx_v.at[pl.ds(toff, tail)]], tb, gsems[0]
            ).wait()
            wait_hbm(cl, 0)
            pltpu.sync_copy(tb, slot(0, tail))
            pltpu.async_copy(
                slot(0, tail), out_hbm.at[pl.ds(base + toff, tail)], hsems[0]
            ).wait()
            start = 1
        else:
            start = 0
        for j in range(start, _NBUF):
            wait_hbm(cl + j, j)

    return gather_kernel


def kernel(idx, table):
    V, D = table.shape
    orig_shape = idx.shape
    idx_flat = idx.reshape(-1).astype(jnp.int32)
    B = idx_flat.shape[0]
    out = _make_gather(V, D, B)(table, idx_flat)
    return out.reshape(*orig_shape, D)


# speedup vs baseline: 1.3312x; 1.0028x over previous
import functools

import jax
import jax.numpy as jnp
from jax import lax
from jax.experimental import pallas as pl
from jax.experimental.pallas import tpu as pltpu
from jax.experimental.pallas import tpu_sc as plsc

_NBUF = 2
_K = 56


def _make_gather(V, D, B):
    info = plsc.get_sparse_core_info()
    NC, NS = info.num_cores, info.num_subcores
    NW = NC * NS
    assert B % NW == 0
    b_per_w = B // NW
    assert b_per_w % 8 == 0
    K = _K
    n_chunks = (b_per_w // K) // _NBUF * _NBUF
    n_rounds = n_chunks // _NBUF
    tail = b_per_w - n_chunks * K
    assert tail % 8 == 0 and tail <= K and n_rounds >= 3

    mesh = plsc.VectorSubcoreMesh(core_axis_name="c", subcore_axis_name="s")

    @functools.partial(
        pl.kernel,
        mesh=mesh,
        out_type=jax.ShapeDtypeStruct((B, D), jnp.float32),
        scratch_types=[
            pltpu.VMEM((b_per_w,), jnp.int32),
            pltpu.VMEM_SHARED((NS * _NBUF * K, D), jnp.float32),
        ]
        + [pltpu.VMEM((K, D), jnp.float32) for _ in range(_NBUF)]
        + [pltpu.SemaphoreType.DMA for _ in range(2 * _NBUF)],
    )
    def gather_kernel(table_hbm, idx_hbm, out_hbm, idx_v, sp, *rest):
        bufs = rest[:_NBUF]
        gsems = rest[_NBUF : 2 * _NBUF]
        hsems = rest[2 * _NBUF :]
        sid = lax.axis_index("s")
        wid = sid * NC + lax.axis_index("c")
        base = wid * b_per_w
        pltpu.sync_copy(idx_hbm.at[pl.ds(base, b_per_w)], idx_v)

        def slot(j, n=K):
            return sp.at[pl.ds((sid * _NBUF + j) * K, n)]

        def start_gather(c, j):
            pltpu.async_copy(
                table_hbm.at[idx_v.at[pl.ds(c * K, K)]], bufs[j], gsems[j]
            )

        def wait_gather(c, j):
            pltpu.make_async_copy(
                table_hbm.at[idx_v.at[pl.ds(c * K, K)]], bufs[j], gsems[j]
            ).wait()

        def start_hbm(c, j):
            pltpu.async_copy(
                slot(j), out_hbm.at[pl.ds(base + c * K, K)], hsems[j]
            )

        def wait_hbm(c, j):
            pltpu.make_async_copy(
                slot(j), out_hbm.at[pl.ds(base + c * K, K)], hsems[j]
            ).wait()

        def step(c, j, first, issue_next):
            wait_gather(c, j)
            if not first:
                wait_hbm(c - _NBUF, j)
            pltpu.sync_copy(bufs[j], slot(j))
            start_hbm(c, j)
            if issue_next:
                start_gather(c + _NBUF, j)

        for j in range(_NBUF):
            start_gather(j, j)
        for j in range(_NBUF):
            step(j, j, True, True)

        def body(i, carry):
            c0 = i * _NBUF
            for j in range(_NBUF):
                step(c0 + j, j, False, True)
            return carry

        lax.fori_loop(1, n_rounds - 1, body, 0)

        cl = (n_rounds - 1) * _NBUF
        for j in range(_NBUF):
            step(cl + j, j, False, False)
        if tail:
            toff = n_chunks * K
            tb = bufs[0].at[pl.ds(0, tail)]
            pltpu.async_copy(
                table_hbm.at[idx_v.at[pl.ds(toff, tail)]], tb, gsems[0]
            ).wait()
            wait_hbm(cl, 0)
            pltpu.sync_copy(tb, slot(0, tail))
            pltpu.async_copy(
                slot(0, tail), out_hbm.at[pl.ds(base + toff, tail)], hsems[0]
            ).wait()
            start = 1
        else:
            start = 0
        for j in range(start, _NBUF):
            wait_hbm(cl + j, j)

    return gather_kernel


def kernel(idx, table):
    V, D = table.shape
    orig_shape = idx.shape
    idx_flat = idx.reshape(-1).astype(jnp.int32)
    B = idx_flat.shape[0]
    out = _make_gather(V, D, B)(table, idx_flat)
    return out.reshape(*orig_shape, D)
